# baseline (speedup 1.0000x reference)
import functools

import jax
import jax.numpy as jnp
from jax import lax
from jax.experimental import pallas as pl
from jax.experimental.pallas import tpu as pltpu
from jax.experimental.pallas import tpu_sc as plsc

NUM_CORES = 2
NUM_SUBCORES = 16
NUM_WORKERS = NUM_CORES * NUM_SUBCORES
LANES = 16


@jax.jit
def _sc_gather_t(idx_t, table):
    hist, batch = idx_t.shape
    vocab, d = table.shape
    assert d == 64 and batch % (NUM_WORKERS * 256) == 0
    bpw = batch // NUM_WORKERS
    halves = bpw // 256
    n_units = hist * halves
    tb_total = batch // 128

    mesh = plsc.VectorSubcoreMesh(
        core_axis_name="c",
        subcore_axis_name="s",
        num_cores=NUM_CORES,
        num_subcores=NUM_SUBCORES,
    )

    @functools.partial(
        pl.kernel,
        out_type=jax.ShapeDtypeStruct((hist, 8, tb_total, 8, 128), jnp.float32),
        mesh=mesh,
        scratch_types=[
            pltpu.VMEM((256,), jnp.int32),
            pltpu.VMEM((256,), jnp.int32),
            pltpu.VMEM((256, 64), jnp.float32),
            pltpu.VMEM((256, 64), jnp.float32),
            pltpu.VMEM((8, 2, 8, 128), jnp.float32),
            pltpu.VMEM((8, 2, 8, 128), jnp.float32),
            pltpu.SemaphoreType.DMA,
            pltpu.SemaphoreType.DMA,
            pltpu.SemaphoreType.DMA,
            pltpu.SemaphoreType.DMA,
            pltpu.SemaphoreType.DMA,
            pltpu.SemaphoreType.DMA,
        ],
        compiler_params=pltpu.CompilerParams(
            use_tc_tiling_on_sc=False, needs_layout_passes=False),
    )
    def k(idx_hbm, table_hbm, out_hbm, idx_v0, idx_v1, rows_v0, rows_v1,
          tr_v0, tr_v1, idx_sem0, idx_sem1, gat_sem0, gat_sem1,
          out_sem0, out_sem1):
        wid = lax.axis_index("s") * NUM_CORES + lax.axis_index("c")
        b_base = wid * bpw

        idx_v = (idx_v0, idx_v1)
        rows_v = (rows_v0, rows_v1)
        tr_v = (tr_v0, tr_v1)
        idx_sem = (idx_sem0, idx_sem1)
        gat_sem = (gat_sem0, gat_sem1)
        out_sem = (out_sem0, out_sem1)

        def unit_hb(u):
            h = u // halves
            b0 = b_base + (u % halves) * 256
            return h, b0

        def start_idx(u, s):
            h, b0 = unit_hb(u)
            pltpu.async_copy(idx_hbm.at[h, pl.ds(b0, 256)], idx_v[s], idx_sem[s])

        def wait_idx(s):
            pltpu.make_async_copy(
                idx_hbm.at[0, pl.ds(0, 256)], idx_v[s], idx_sem[s]).wait()

        def start_gather(s):
            pltpu.async_copy(table_hbm.at[idx_v[s]], rows_v[s], gat_sem[s])

        def wait_gather(s):
            pltpu.make_async_copy(
                table_hbm.at[idx_v[s]], rows_v[s], gat_sem[s]).wait()

        def start_store(u, s):
            h, b0 = unit_hb(u)
            tbq = (b0 - b_base) // 128 + wid * (bpw // 128)
            pltpu.async_copy(
                tr_v[s], out_hbm.at[h, :, pl.ds(tbq, 2), :, :], out_sem[s])

        def wait_store(s):
            pltpu.make_async_copy(
                tr_v[s], out_hbm.at[0, :, pl.ds(0, 2), :, :], out_sem[s]).wait()

        iota = lax.iota(jnp.int32, LANES)

        def transpose(s):
            rows = rows_v[s]
            dst = tr_v[s]

            def body(dd):
                tr = dd // 8
                sb = dd % 8
                col = jnp.full((LANES,), dd, jnp.int32)
                for tj in range(2):
                    for lb in range(8):
                        rix = iota + (tj * 128 + lb * 16)
                        v = plsc.load_gather(rows, [rix, col])
                        dst[tr, tj, sb, pl.ds(lb * 16, LANES)] = v

            pl.loop(0, 64)(body)

        start_idx(0, 0)
        start_idx(1, 1)
        wait_idx(0)
        start_gather(0)

        def unit_body(u, s):
            o = 1 - s
            wait_gather(s)
            pl.when(u + 2 < n_units)(lambda: start_idx(u + 2, s))
            def launch_next():
                wait_idx(o)
                start_gather(o)
            pl.when(u + 1 < n_units)(launch_next)
            pl.when(u >= 2)(lambda: wait_store(s))
            transpose(s)
            start_store(u, s)

        def outer(i):
            unit_body(i * 2, 0)
            unit_body(i * 2 + 1, 1)

        pl.loop(0, n_units // 2)(outer)

        wait_store(0)
        wait_store(1)

    return k(idx_t, table)


def kernel(idxs, entity_embeds):
    batch, hist = idxs.shape
    d = entity_embeds.shape[1]
    out_t = _sc_gather_t(idxs.T, entity_embeds)
    return out_t.transpose(2, 4, 0, 1, 3).reshape(batch, hist, d)


# speedup vs baseline: 4.3324x; 4.3324x over previous
import functools

import jax
import jax.numpy as jnp
from jax import lax
from jax.experimental import pallas as pl
from jax.experimental.pallas import tpu as pltpu
from jax.experimental.pallas import tpu_sc as plsc

NUM_CORES = 2
NUM_SUBCORES = 16
NUM_WORKERS = NUM_CORES * NUM_SUBCORES
LANES = 16


@jax.jit
def _sc_gather_t(idx_t, table):
    hist, batch = idx_t.shape
    vocab, d = table.shape
    assert d == 64 and batch % (NUM_WORKERS * 256) == 0
    bpw = batch // NUM_WORKERS
    halves = bpw // 256
    n_units = hist * halves
    tb_total = batch // 128

    mesh = plsc.VectorSubcoreMesh(
        core_axis_name="c",
        subcore_axis_name="s",
        num_cores=NUM_CORES,
        num_subcores=NUM_SUBCORES,
    )

    @functools.partial(
        pl.kernel,
        out_type=jax.ShapeDtypeStruct((hist, 8, tb_total, 8, 128), jnp.float32),
        mesh=mesh,
        scratch_types=[
            pltpu.VMEM((256,), jnp.int32),
            pltpu.VMEM((256,), jnp.int32),
            pltpu.VMEM((256, 64), jnp.float32),
            pltpu.VMEM((256, 64), jnp.float32),
            pltpu.VMEM((8, 2, 8, 128), jnp.float32),
            pltpu.VMEM((8, 2, 8, 128), jnp.float32),
            pltpu.SemaphoreType.DMA,
            pltpu.SemaphoreType.DMA,
            pltpu.SemaphoreType.DMA,
            pltpu.SemaphoreType.DMA,
            pltpu.SemaphoreType.DMA,
            pltpu.SemaphoreType.DMA,
        ],
        compiler_params=pltpu.CompilerParams(
            use_tc_tiling_on_sc=False, needs_layout_passes=False),
    )
    def k(idx_hbm, table_hbm, out_hbm, idx_v0, idx_v1, rows_v0, rows_v1,
          tr_v0, tr_v1, idx_sem0, idx_sem1, gat_sem0, gat_sem1,
          out_sem0, out_sem1):
        wid = lax.axis_index("s") * NUM_CORES + lax.axis_index("c")
        b_base = wid * bpw

        idx_v = (idx_v0, idx_v1)
        rows_v = (rows_v0, rows_v1)
        tr_v = (tr_v0, tr_v1)
        idx_sem = (idx_sem0, idx_sem1)
        gat_sem = (gat_sem0, gat_sem1)
        out_sem = (out_sem0, out_sem1)

        def unit_hb(u):
            h = u // halves
            b0 = b_base + (u % halves) * 256
            return h, b0

        def start_idx(u, s):
            h, b0 = unit_hb(u)
            pltpu.async_copy(idx_hbm.at[h, pl.ds(b0, 256)], idx_v[s], idx_sem[s])

        def wait_idx(s):
            pltpu.make_async_copy(
                idx_hbm.at[0, pl.ds(0, 256)], idx_v[s], idx_sem[s]).wait()

        def start_gather(s):
            pltpu.async_copy(table_hbm.at[idx_v[s]], rows_v[s], gat_sem[s])

        def wait_gather(s):
            pltpu.make_async_copy(
                table_hbm.at[idx_v[s]], rows_v[s], gat_sem[s]).wait()

        def start_store(u, s):
            h, b0 = unit_hb(u)
            tbq = (b0 - b_base) // 128 + wid * (bpw // 128)
            pltpu.async_copy(
                tr_v[s], out_hbm.at[h, :, pl.ds(tbq, 2), :, :], out_sem[s])

        def wait_store(s):
            pltpu.make_async_copy(
                tr_v[s], out_hbm.at[0, :, pl.ds(0, 2), :, :], out_sem[s]).wait()

        iota = lax.iota(jnp.int32, LANES)
        lvecs = [iota + 16 * lb for lb in range(8)]
        rvecs = [[iota + 16 * lb + 128 * tj for lb in range(8)]
                 for tj in range(2)]
        cvecs = [lax.bitwise_and(iota + c0, 15) for c0 in range(16)]
        tjcs = [jnp.full((LANES,), tj, jnp.int32) for tj in range(2)]

        def transpose(s):
            rows = rows_v[s]
            dst = tr_v[s]

            def group(u):
                cb = u // 16
                c0 = u - cb * 16
                col = lax.bitwise_and(iota + c0, 15) + cb * 16
                trv = lax.shift_right_logical(col, 3)
                sbv = lax.bitwise_and(col, 7)
                vals = []
                for tj in range(2):
                    for lb in range(8):
                        vals.append(
                            plsc.load_gather(rows, [rvecs[tj][lb], col]))
                for tj in range(2):
                    for lb in range(8):
                        plsc.store_scatter(
                            dst, [trv, tjcs[tj], sbv, lvecs[lb]],
                            vals[tj * 8 + lb])

            pl.loop(0, 64)(group)

        start_idx(0, 0)
        start_idx(1, 1)
        wait_idx(0)
        start_gather(0)

        def unit_body(u, s):
            o = 1 - s
            wait_gather(s)
            pl.when(u + 2 < n_units)(lambda: start_idx(u + 2, s))
            def launch_next():
                wait_idx(o)
                start_gather(o)
            pl.when(u + 1 < n_units)(launch_next)
            pl.when(u >= 2)(lambda: wait_store(s))
            transpose(s)
            start_store(u, s)

        def outer(i):
            unit_body(i * 2, 0)
            unit_body(i * 2 + 1, 1)

        pl.loop(0, n_units // 2)(outer)

        wait_store(0)
        wait_store(1)

    return k(idx_t, table)


def kernel(idxs, entity_embeds):
    batch, hist = idxs.shape
    d = entity_embeds.shape[1]
    out_t = _sc_gather_t(idxs.T, entity_embeds)
    return out_t.transpose(2, 4, 0, 1, 3).reshape(batch, hist, d)
